# Initial kernel scaffold; baseline (speedup 1.0000x reference)
import jax
import jax.numpy as jnp
from jax.experimental import pallas as pl


def kernel(x, W16, W32, W64):
    raise NotImplementedError("write your pallas kernel here")


# speedup vs baseline: 1.1444x; 1.1444x over previous
import functools

import jax
import jax.numpy as jnp
from jax import lax
from jax.experimental import pallas as pl
from jax.experimental.pallas import tpu as pltpu
from jax.experimental.pallas import tpu_sc as plsc

_DIMS = ([16, 32, 64] * 8) + [16, 32]
_VOCAB = 100000
_BATCH = 16384
_NC = 2
_NS = 16
_NW = _NC * _NS
_BPW = _BATCH // _NW
_CHUNK = 128
_NCHUNK = _BPW // _CHUNK


def _field_offsets():
    counters = {16: 0, 32: 0, 64: 0}
    offs = []
    for d in _DIMS:
        offs.append(counters[d] * _VOCAB)
        counters[d] += 1
    return offs


@functools.partial(jax.jit, static_argnums=())
def kernel(x, W16, W32, W64):
    f16 = W16.reshape(9 * _VOCAB, 16)
    f32 = W32.reshape(9 * _VOCAB, 32)
    f64 = W64.reshape(8 * _VOCAB, 64)
    offs = jnp.asarray(_field_offsets(), dtype=jnp.int32)
    xt = (x + offs[None, :]).T.reshape(26, _BATCH // _CHUNK, _CHUNK)

    mesh = plsc.VectorSubcoreMesh(core_axis_name="c", subcore_axis_name="s")
    out_type = tuple(
        jax.ShapeDtypeStruct((_BATCH, d), jnp.float32) for d in _DIMS
    )

    @functools.partial(
        pl.kernel,
        mesh=mesh,
        out_type=out_type,
        compiler_params=pltpu.CompilerParams(use_tc_tiling_on_sc=False),
        scratch_types=[
            pltpu.VMEM((_NCHUNK, _CHUNK), jnp.int32),
            pltpu.VMEM((_BPW, 16), jnp.float32),
            pltpu.VMEM((_BPW, 32), jnp.float32),
            pltpu.VMEM((_BPW, 64), jnp.float32),
            pltpu.SemaphoreType.DMA,
        ],
    )
    def run(xt_hbm, t16, t32, t64, *rest):
        outs = rest[:26]
        idx_v, r16, r32, r64, sem = rest[26:]
        tabs = {16: t16, 32: t32, 64: t64}
        bufs = {16: r16, 32: r32, 64: r64}
        wid = lax.axis_index("s") * _NC + lax.axis_index("c")
        base = wid * _BPW
        cbase = wid * _NCHUNK
        for f in range(26):
            d = _DIMS[f]
            tab, rows = tabs[d], bufs[d]
            pltpu.sync_copy(xt_hbm.at[f, pl.ds(cbase, _NCHUNK)], idx_v)
            copies = [
                pltpu.async_copy(
                    tab.at[idx_v.at[j]],
                    rows.at[pl.ds(j * _CHUNK, _CHUNK)],
                    sem,
                )
                for j in range(_NCHUNK)
            ]
            for c in copies:
                c.wait()
            pltpu.sync_copy(rows, outs[f].at[pl.ds(base, _BPW)])

    return run(xt, f16, f32, f64)


# speedup vs baseline: 1.1667x; 1.0195x over previous
import functools

import jax
import jax.numpy as jnp
from jax import lax
from jax.experimental import pallas as pl
from jax.experimental.pallas import tpu as pltpu
from jax.experimental.pallas import tpu_sc as plsc

_DIMS = ([16, 32, 64] * 8) + [16, 32]
_VOCAB = 100000
_BATCH = 16384
_NC = 2
_NS = 16
_NW = _NC * _NS
_BPW = _BATCH // _NW
_CHUNK = 128
_NCHUNK = _BPW // _CHUNK


def _field_offsets():
    counters = {16: 0, 32: 0, 64: 0}
    offs = []
    for d in _DIMS:
        offs.append(counters[d] * _VOCAB)
        counters[d] += 1
    return offs


@functools.partial(jax.jit, static_argnums=())
def kernel(x, W16, W32, W64):
    f16 = W16.reshape(9 * _VOCAB, 16)
    f32 = W32.reshape(9 * _VOCAB, 32)
    f64 = W64.reshape(8 * _VOCAB, 64)
    offs = jnp.asarray(_field_offsets(), dtype=jnp.int32)
    xt = (x + offs[None, :]).T.reshape(26, _NW, _NCHUNK, _CHUNK)
    xw = xt.transpose(1, 0, 2, 3).reshape(_NW, 26 * _NCHUNK, _CHUNK)

    mesh = plsc.VectorSubcoreMesh(core_axis_name="c", subcore_axis_name="s")
    out_type = tuple(
        jax.ShapeDtypeStruct((_BATCH, d), jnp.float32) for d in _DIMS
    )

    @functools.partial(
        pl.kernel,
        mesh=mesh,
        out_type=out_type,
        compiler_params=pltpu.CompilerParams(use_tc_tiling_on_sc=False),
        scratch_types=[
            pltpu.VMEM((26 * _NCHUNK, _CHUNK), jnp.int32),
            pltpu.VMEM((_BPW, 16), jnp.float32),
            pltpu.VMEM((_BPW, 32), jnp.float32),
            pltpu.VMEM((_BPW, 64), jnp.float32),
            pltpu.SemaphoreType.DMA,
            pltpu.SemaphoreType.DMA,
        ],
    )
    def run(xw_hbm, t16, t32, t64, *rest):
        outs = rest[:26]
        idx_v, r16, r32, r64, gsem, wsem = rest[26:]
        tabs = {16: t16, 32: t32, 64: t64}
        bufs = {16: r16, 32: r32, 64: r64}
        wid = lax.axis_index("s") * _NC + lax.axis_index("c")
        base = wid * _BPW
        pltpu.sync_copy(xw_hbm.at[wid], idx_v)

        def fire(f):
            d = _DIMS[f]
            return [
                pltpu.async_copy(
                    tabs[d].at[idx_v.at[f * _NCHUNK + j]],
                    bufs[d].at[pl.ds(j * _CHUNK, _CHUNK)],
                    gsem,
                )
                for j in range(_NCHUNK)
            ]

        writeback = {16: None, 32: None, 64: None}
        inflight = fire(0)
        for f in range(26):
            d = _DIMS[f]
            if f + 1 < 26:
                dn = _DIMS[f + 1]
                if writeback[dn] is not None:
                    writeback[dn].wait()
                    writeback[dn] = None
                nxt = fire(f + 1)
            for c in inflight:
                c.wait()
            writeback[d] = pltpu.async_copy(
                bufs[d], outs[f].at[pl.ds(base, _BPW)], wsem
            )
            if f + 1 < 26:
                inflight = nxt
        for d in (16, 32, 64):
            if writeback[d] is not None:
                writeback[d].wait()

    return run(xw, f16, f32, f64)
